# Initial kernel scaffold; baseline (speedup 1.0000x reference)
import jax
import jax.numpy as jnp
from jax.experimental import pallas as pl


def kernel(input_ids, embedding_table, hard_norm_weight, proj_weight):
    raise NotImplementedError("write your pallas kernel here")


# speedup vs baseline: 1.8736x; 1.8736x over previous
import functools

import jax
import jax.numpy as jnp
from jax import lax
from jax.experimental import pallas as pl
from jax.experimental.pallas import tpu as pltpu
from jax.experimental.pallas import tpu_sc as plsc

EPS = 1e-06


def _sc_gather(table, idx, n_tokens, mm_dim, nw, chunk):
    n_chunks = n_tokens // (nw * chunk)
    mesh = plsc.VectorSubcoreMesh(core_axis_name="c", subcore_axis_name="s")
    nc = mesh.num_cores

    @functools.partial(
        pl.kernel,
        out_type=jax.ShapeDtypeStruct((n_tokens, mm_dim), jnp.float32),
        mesh=mesh,
        scratch_types=[
            pltpu.VMEM((n_chunks, chunk), jnp.int32),
            pltpu.VMEM((chunk, mm_dim), jnp.float32),
            pltpu.SemaphoreType.DMA,
        ],
    )
    def gather_kernel(table_hbm, idx_hbm, out_hbm, idx_v, rows_v, sem):
        wid = lax.axis_index("s") * nc + lax.axis_index("c")
        base = wid * n_chunks * chunk
        pltpu.sync_copy(idx_hbm.at[wid], idx_v)
        for c in range(n_chunks):
            pltpu.async_copy(table_hbm.at[idx_v.at[c]], rows_v, sem).wait()
            pltpu.sync_copy(rows_v, out_hbm.at[pl.ds(base + c * chunk, chunk)])

    return gather_kernel(table, idx)


def _tc_norm_proj_norm(emb, hw, w_bf16, n_tokens, mm_dim, txt_dim, blk):
    def body(x_ref, hw_ref, w_ref, o_ref):
        x = x_ref[...]
        var = jnp.mean(x * x, axis=-1, keepdims=True)
        xn = (x * lax.rsqrt(var + EPS) * hw_ref[...]).astype(jnp.bfloat16)
        y = lax.dot_general(
            xn, w_ref[...], (((1,), (1,)), ((), ())),
            preferred_element_type=jnp.float32,
        )
        var2 = jnp.mean(y * y, axis=-1, keepdims=True)
        o_ref[...] = y * lax.rsqrt(var2 + EPS)

    return pl.pallas_call(
        body,
        grid=(n_tokens // blk,),
        in_specs=[
            pl.BlockSpec((blk, mm_dim), lambda i: (i, 0)),
            pl.BlockSpec((1, mm_dim), lambda i: (0, 0)),
            pl.BlockSpec((txt_dim, mm_dim), lambda i: (0, 0)),
        ],
        out_specs=pl.BlockSpec((blk, txt_dim), lambda i: (i, 0)),
        out_shape=jax.ShapeDtypeStruct((n_tokens, txt_dim), jnp.float32),
    )(emb, hw, w_bf16)


def kernel(input_ids, embedding_table, hard_norm_weight, proj_weight):
    b, s = input_ids.shape
    vocab, mm_dim = embedding_table.shape
    txt_dim = proj_weight.shape[0]
    n_tokens = b * s

    nw = 32
    chunk = 64
    idx = input_ids.reshape(nw, n_tokens // (nw * chunk), chunk).astype(jnp.int32)

    emb = _sc_gather(embedding_table, idx, n_tokens, mm_dim, nw, chunk)
    out = _tc_norm_proj_norm(
        emb,
        hard_norm_weight.reshape(1, mm_dim),
        proj_weight.astype(jnp.bfloat16),
        n_tokens, mm_dim, txt_dim, blk=256,
    )
    return out.reshape(b, s, txt_dim)


# speedup vs baseline: 2.1559x; 1.1507x over previous
import functools

import jax
import jax.numpy as jnp
from jax import lax
from jax.experimental import pallas as pl
from jax.experimental.pallas import tpu as pltpu
from jax.experimental.pallas import tpu_sc as plsc

EPS = 1e-06


def _sc_gather(table, idx, n_tokens, mm_dim, nw, chunk):
    n_chunks = n_tokens // (nw * chunk)
    mesh = plsc.VectorSubcoreMesh(core_axis_name="c", subcore_axis_name="s")
    nc = mesh.num_cores

    @functools.partial(
        pl.kernel,
        out_type=jax.ShapeDtypeStruct((n_tokens, mm_dim), jnp.float32),
        mesh=mesh,
        scratch_types=[
            pltpu.VMEM((n_chunks, chunk), jnp.int32),
            pltpu.VMEM((chunk, mm_dim), jnp.float32),
            pltpu.VMEM((chunk, mm_dim), jnp.float32),
            pltpu.SemaphoreType.DMA,
            pltpu.SemaphoreType.DMA,
        ],
    )
    def gather_kernel(table_hbm, idx_hbm, out_hbm, idx_v, rows_a, rows_b, g_sem, o_sem):
        wid = lax.axis_index("s") * nc + lax.axis_index("c")
        base = wid * n_chunks * chunk
        bufs = (rows_a, rows_b)
        pltpu.sync_copy(idx_hbm.at[wid], idx_v)
        pltpu.async_copy(table_hbm.at[idx_v.at[0]], bufs[0], g_sem)
        for c in range(n_chunks):
            buf = bufs[c % 2]
            pltpu.make_async_copy(table_hbm.at[idx_v.at[c]], buf, g_sem).wait()
            if c + 1 < n_chunks:
                if c >= 1:
                    pltpu.make_async_copy(
                        bufs[(c + 1) % 2],
                        out_hbm.at[pl.ds(base + (c - 1) * chunk, chunk)],
                        o_sem).wait()
                pltpu.async_copy(
                    table_hbm.at[idx_v.at[c + 1]], bufs[(c + 1) % 2], g_sem)
            pltpu.async_copy(
                buf, out_hbm.at[pl.ds(base + c * chunk, chunk)], o_sem)
        for c in range(max(n_chunks - 2, 0), n_chunks):
            pltpu.make_async_copy(
                bufs[c % 2],
                out_hbm.at[pl.ds(base + c * chunk, chunk)],
                o_sem).wait()

    return gather_kernel(table, idx)


def _tc_norm_proj_norm(emb, hw, w_bf16, n_tokens, mm_dim, txt_dim, blk):
    def body(x_ref, hw_ref, w_ref, o_ref):
        x = x_ref[...]
        var = jnp.mean(x * x, axis=-1, keepdims=True)
        xn = (x * lax.rsqrt(var + EPS) * hw_ref[...]).astype(jnp.bfloat16)
        y = lax.dot_general(
            xn, w_ref[...], (((1,), (1,)), ((), ())),
            preferred_element_type=jnp.float32,
        )
        var2 = jnp.mean(y * y, axis=-1, keepdims=True)
        o_ref[...] = y * lax.rsqrt(var2 + EPS)

    return pl.pallas_call(
        body,
        grid=(n_tokens // blk,),
        in_specs=[
            pl.BlockSpec((blk, mm_dim), lambda i: (i, 0)),
            pl.BlockSpec((1, mm_dim), lambda i: (0, 0)),
            pl.BlockSpec((txt_dim, mm_dim), lambda i: (0, 0)),
        ],
        out_specs=pl.BlockSpec((blk, txt_dim), lambda i: (i, 0)),
        out_shape=jax.ShapeDtypeStruct((n_tokens, txt_dim), jnp.float32),
    )(emb, hw, w_bf16)


def kernel(input_ids, embedding_table, hard_norm_weight, proj_weight):
    b, s = input_ids.shape
    vocab, mm_dim = embedding_table.shape
    txt_dim = proj_weight.shape[0]
    n_tokens = b * s

    nw = 32
    chunk = 32
    idx = input_ids.reshape(nw, n_tokens // (nw * chunk), chunk).astype(jnp.int32)

    emb = _sc_gather(embedding_table, idx, n_tokens, mm_dim, nw, chunk)
    out = _tc_norm_proj_norm(
        emb,
        hard_norm_weight.reshape(1, mm_dim),
        proj_weight.astype(jnp.bfloat16),
        n_tokens, mm_dim, txt_dim, blk=512,
    )
    return out.reshape(b, s, txt_dim)
